# baseline (device time: 60927 ns/iter reference)
import jax
import jax.numpy as jnp
from jax import lax
from jax.experimental import pallas as pl
from jax.experimental.pallas import tpu as pltpu

M = 1024
D = 1024
N_GROUPS = 4
ROWS = M // N_GROUPS
EPS = 1e-6


def kernel(partial, resid, gamma):
    partial2d = partial.reshape(M, D)
    gamma2d = gamma.reshape(1, D)

    def body(p_ref, r_ref, g_ref, out_ref, ycomm, send_sems, recv_sems):
        my_x = lax.axis_index("x")
        my_y = lax.axis_index("y")
        my_z = lax.axis_index("z")
        xp = (1 - my_x, my_y, my_z)
        yp = (my_x, 1 - my_y, my_z)
        zp = (my_x, my_y, 1 - my_z)

        grp = 2 * my_x + my_z
        row0 = grp * ROWS

        barrier = pltpu.get_barrier_semaphore()
        for nbr in (xp, yp, zp):
            pl.semaphore_signal(
                barrier, inc=1, device_id=nbr,
                device_id_type=pl.DeviceIdType.MESH,
            )
        pl.semaphore_wait(barrier, 3)

        p1 = pltpu.make_async_remote_copy(
            src_ref=p_ref.at[pl.ds(row0, ROWS), :],
            dst_ref=ycomm,
            send_sem=send_sems.at[0],
            recv_sem=recv_sems.at[0],
            device_id=yp,
            device_id_type=pl.DeviceIdType.MESH,
        )
        p1.start()
        p1.wait()

        acc = (
            p_ref[pl.ds(row0, ROWS), :]
            + ycomm[...]
            + r_ref[pl.ds(row0, ROWS), :]
        )
        rms = jnp.sqrt(jnp.mean(acc * acc, axis=-1, keepdims=True) + EPS)
        out_ref[pl.ds(row0, ROWS), :] = acc / rms * g_ref[...]

        p2 = pltpu.make_async_remote_copy(
            src_ref=out_ref.at[pl.ds(row0, ROWS), :],
            dst_ref=out_ref.at[pl.ds(row0, ROWS), :],
            send_sem=send_sems.at[1],
            recv_sem=recv_sems.at[1],
            device_id=xp,
            device_id_type=pl.DeviceIdType.MESH,
        )
        p2.start()
        p2.wait()

        a0 = my_z * ROWS
        a1 = (2 + my_z) * ROWS
        p3a = pltpu.make_async_remote_copy(
            src_ref=out_ref.at[pl.ds(a0, ROWS), :],
            dst_ref=out_ref.at[pl.ds(a0, ROWS), :],
            send_sem=send_sems.at[2],
            recv_sem=recv_sems.at[2],
            device_id=zp,
            device_id_type=pl.DeviceIdType.MESH,
        )
        p3b = pltpu.make_async_remote_copy(
            src_ref=out_ref.at[pl.ds(a1, ROWS), :],
            dst_ref=out_ref.at[pl.ds(a1, ROWS), :],
            send_sem=send_sems.at[3],
            recv_sem=recv_sems.at[3],
            device_id=zp,
            device_id_type=pl.DeviceIdType.MESH,
        )
        p3a.start()
        p3b.start()
        p3a.wait()
        p3b.wait()

    return pl.pallas_call(
        body,
        out_shape=jax.ShapeDtypeStruct((M, D), jnp.float32),
        in_specs=[
            pl.BlockSpec(memory_space=pltpu.VMEM),
            pl.BlockSpec(memory_space=pltpu.VMEM),
            pl.BlockSpec(memory_space=pltpu.VMEM),
        ],
        out_specs=pl.BlockSpec(memory_space=pltpu.VMEM),
        scratch_shapes=[
            pltpu.VMEM((ROWS, D), jnp.float32),
            pltpu.SemaphoreType.DMA((4,)),
            pltpu.SemaphoreType.DMA((4,)),
        ],
        compiler_params=pltpu.CompilerParams(collective_id=0),
    )(partial2d, resid, gamma2d)


# device time: 40020 ns/iter; 1.5224x vs baseline; 1.5224x over previous
import jax
import jax.numpy as jnp
from jax import lax
from jax.experimental import pallas as pl
from jax.experimental.pallas import tpu as pltpu

M = 1024
D = 1024
N_GROUPS = 4
ROWS = M // N_GROUPS
N_TILES = 4
TILE = ROWS // N_TILES
EPS = 1e-6


def kernel(partial, resid, gamma):
    partial2d = partial.reshape(M, D)
    gamma2d = gamma.reshape(1, D)

    def body(p_ref, r_ref, g_ref, out_ref, ycomm, send_sems, recv_sems):
        my_x = lax.axis_index("x")
        my_y = lax.axis_index("y")
        my_z = lax.axis_index("z")
        xp = (1 - my_x, my_y, my_z)
        yp = (my_x, 1 - my_y, my_z)
        zp = (my_x, my_y, 1 - my_z)

        row0 = (2 * my_x + my_z) * ROWS
        xrow0 = (2 * (1 - my_x) + my_z) * ROWS
        zrow0 = (2 * my_x + (1 - my_z)) * ROWS

        def exchange(src, dst, sem, peer):
            return pltpu.make_async_remote_copy(
                src_ref=src, dst_ref=dst,
                send_sem=send_sems.at[sem], recv_sem=recv_sems.at[sem],
                device_id=peer, device_id_type=pl.DeviceIdType.MESH,
            )

        barrier = pltpu.get_barrier_semaphore()
        for nbr in (xp, yp, zp):
            pl.semaphore_signal(
                barrier, inc=1, device_id=nbr,
                device_id_type=pl.DeviceIdType.MESH,
            )
        pl.semaphore_wait(barrier, 3)

        yex = []
        for t in range(N_TILES):
            r = exchange(
                p_ref.at[pl.ds(row0 + t * TILE, TILE), :],
                ycomm.at[pl.ds(t * TILE, TILE), :],
                t, yp,
            )
            r.start()
            yex.append(r)

        hop1 = []
        for t in range(N_TILES):
            yex[t].wait()
            rows = pl.ds(row0 + t * TILE, TILE)
            acc = p_ref[rows, :] + ycomm[pl.ds(t * TILE, TILE), :] + r_ref[rows, :]
            rms = jnp.sqrt(jnp.mean(acc * acc, axis=-1, keepdims=True) + EPS)
            out_ref[rows, :] = acc / rms * g_ref[...]
            peer = xp if t % 2 == 0 else zp
            r = exchange(out_ref.at[rows, :], out_ref.at[rows, :], 4 + t, peer)
            r.start()
            hop1.append(r)

        hop2 = []
        for t in range(N_TILES):
            hop1[t].wait()
            recv_row0 = xrow0 if t % 2 == 0 else zrow0
            peer = zp if t % 2 == 0 else xp
            mine = pl.ds(row0 + t * TILE, TILE)
            got = pl.ds(recv_row0 + t * TILE, TILE)
            ra = exchange(out_ref.at[mine, :], out_ref.at[mine, :], 8 + t, peer)
            rb = exchange(out_ref.at[got, :], out_ref.at[got, :], 12 + t, peer)
            ra.start()
            rb.start()
            hop2.append((ra, rb))

        for ra, rb in hop2:
            ra.wait()
            rb.wait()

    return pl.pallas_call(
        body,
        out_shape=jax.ShapeDtypeStruct((M, D), jnp.float32),
        in_specs=[
            pl.BlockSpec(memory_space=pltpu.VMEM),
            pl.BlockSpec(memory_space=pltpu.VMEM),
            pl.BlockSpec(memory_space=pltpu.VMEM),
        ],
        out_specs=pl.BlockSpec(memory_space=pltpu.VMEM),
        scratch_shapes=[
            pltpu.VMEM((ROWS, D), jnp.float32),
            pltpu.SemaphoreType.DMA((16,)),
            pltpu.SemaphoreType.DMA((16,)),
        ],
        compiler_params=pltpu.CompilerParams(collective_id=0),
    )(partial2d, resid, gamma2d)


# device time: 39429 ns/iter; 1.5452x vs baseline; 1.0150x over previous
import jax
import jax.numpy as jnp
from jax import lax
from jax.experimental import pallas as pl
from jax.experimental.pallas import tpu as pltpu

M = 1024
D = 1024
N_GROUPS = 4
ROWS = M // N_GROUPS
N_TILES = 8
TILE = ROWS // N_TILES
EPS = 1e-6


def kernel(partial, resid, gamma):
    partial2d = partial.reshape(M, D)
    gamma2d = gamma.reshape(1, D)

    def body(p_ref, r_ref, g_ref, out_ref, pvm, rvm, ycomm, vmine, vin,
             send_sems, recv_sems, local_sems):
        my_x = lax.axis_index("x")
        my_y = lax.axis_index("y")
        my_z = lax.axis_index("z")
        xp = (1 - my_x, my_y, my_z)
        yp = (my_x, 1 - my_y, my_z)
        zp = (my_x, my_y, 1 - my_z)

        row0 = (2 * my_x + my_z) * ROWS
        xrow0 = (2 * (1 - my_x) + my_z) * ROWS
        zrow0 = (2 * my_x + (1 - my_z)) * ROWS

        def exchange(src, dst, sem, peer):
            return pltpu.make_async_remote_copy(
                src_ref=src, dst_ref=dst,
                send_sem=send_sems.at[sem], recv_sem=recv_sems.at[sem],
                device_id=peer, device_id_type=pl.DeviceIdType.MESH,
            )

        cp_p = pltpu.make_async_copy(
            p_ref.at[pl.ds(row0, ROWS), :], pvm, local_sems.at[0])
        cp_r = pltpu.make_async_copy(
            r_ref.at[pl.ds(row0, ROWS), :], rvm, local_sems.at[1])
        cp_p.start()
        cp_r.start()

        barrier = pltpu.get_barrier_semaphore()
        for nbr in (xp, yp, zp):
            pl.semaphore_signal(
                barrier, inc=1, device_id=nbr,
                device_id_type=pl.DeviceIdType.MESH,
            )
        pl.semaphore_wait(barrier, 3)

        cp_p.wait()
        yex = []
        for t in range(N_TILES):
            tl = pl.ds(t * TILE, TILE)
            r = exchange(pvm.at[tl, :], ycomm.at[tl, :], t, yp)
            r.start()
            yex.append(r)
        cp_r.wait()

        hop1 = []
        stores = []
        for t in range(N_TILES):
            yex[t].wait()
            tl = pl.ds(t * TILE, TILE)
            acc = pvm[tl, :] + ycomm[tl, :] + rvm[tl, :]
            rms = jnp.sqrt(jnp.mean(acc * acc, axis=-1, keepdims=True) + EPS)
            vmine[tl, :] = acc / rms * g_ref[...]
            st = pltpu.make_async_copy(
                vmine.at[tl, :],
                out_ref.at[pl.ds(row0 + t * TILE, TILE), :],
                local_sems.at[2 + t],
            )
            st.start()
            stores.append(st)
            peer = xp if t % 2 == 0 else zp
            r = exchange(vmine.at[tl, :], vin.at[tl, :], 8 + t, peer)
            r.start()
            hop1.append(r)

        hop2 = []
        for t in range(N_TILES):
            hop1[t].wait()
            tl = pl.ds(t * TILE, TILE)
            recv_row0 = xrow0 if t % 2 == 0 else zrow0
            peer = zp if t % 2 == 0 else xp
            mine_rows = pl.ds(row0 + t * TILE, TILE)
            got_rows = pl.ds(recv_row0 + t * TILE, TILE)
            st = pltpu.make_async_copy(
                vin.at[tl, :], out_ref.at[got_rows, :],
                local_sems.at[10 + t],
            )
            st.start()
            stores.append(st)
            ra = exchange(vmine.at[tl, :], out_ref.at[mine_rows, :],
                          16 + t, peer)
            rb = exchange(vin.at[tl, :], out_ref.at[got_rows, :],
                          24 + t, peer)
            ra.start()
            rb.start()
            hop2.append((ra, rb))

        for ra, rb in hop2:
            ra.wait()
            rb.wait()
        for st in stores:
            st.wait()

    return pl.pallas_call(
        body,
        out_shape=jax.ShapeDtypeStruct((M, D), jnp.float32),
        in_specs=[
            pl.BlockSpec(memory_space=pl.ANY),
            pl.BlockSpec(memory_space=pl.ANY),
            pl.BlockSpec(memory_space=pltpu.VMEM),
        ],
        out_specs=pl.BlockSpec(memory_space=pl.ANY),
        scratch_shapes=[
            pltpu.VMEM((ROWS, D), jnp.float32),
            pltpu.VMEM((ROWS, D), jnp.float32),
            pltpu.VMEM((ROWS, D), jnp.float32),
            pltpu.VMEM((ROWS, D), jnp.float32),
            pltpu.VMEM((ROWS, D), jnp.float32),
            pltpu.SemaphoreType.DMA((32,)),
            pltpu.SemaphoreType.DMA((32,)),
            pltpu.SemaphoreType.DMA((18,)),
        ],
        compiler_params=pltpu.CompilerParams(collective_id=0),
    )(partial2d, resid, gamma2d)


# device time: 32419 ns/iter; 1.8794x vs baseline; 1.2162x over previous
import jax
import jax.numpy as jnp
from jax import lax
from jax.experimental import pallas as pl
from jax.experimental.pallas import tpu as pltpu

M = 1024
D = 1024
N_GROUPS = 4
ROWS = M // N_GROUPS
N_TILES = 16
TILE = ROWS // N_TILES
S0 = 10
NS = N_TILES - S0
EPS = 1e-6


def kernel(partial, resid, gamma):
    partial2d = partial.reshape(M, D)
    gamma2d = gamma.reshape(1, D)

    def body(p_ref, r_ref, g_ref, out_ref, pvm, rvm, ycomm,
             pvd, rvd, ycd, vmine, vdiag, vin,
             send_sems, recv_sems, local_sems):
        my_x = lax.axis_index("x")
        my_y = lax.axis_index("y")
        my_z = lax.axis_index("z")
        xp = (1 - my_x, my_y, my_z)
        yp = (my_x, 1 - my_y, my_z)
        zp = (my_x, my_y, 1 - my_z)

        row0 = (2 * my_x + my_z) * ROWS
        xrow0 = (2 * (1 - my_x) + my_z) * ROWS
        zrow0 = (2 * my_x + (1 - my_z)) * ROWS
        drow0 = (2 * (1 - my_x) + (1 - my_z)) * ROWS

        def exchange(src, dst, sem, peer):
            return pltpu.make_async_remote_copy(
                src_ref=src, dst_ref=dst,
                send_sem=send_sems.at[sem], recv_sem=recv_sems.at[sem],
                device_id=peer, device_id_type=pl.DeviceIdType.MESH,
            )

        def rms_norm(acc):
            rms = jnp.sqrt(jnp.mean(acc * acc, axis=-1, keepdims=True) + EPS)
            return acc / rms * g_ref[...]

        cp_p = pltpu.make_async_copy(
            p_ref.at[pl.ds(row0, ROWS), :], pvm, local_sems.at[0])
        cp_r = pltpu.make_async_copy(
            r_ref.at[pl.ds(row0, ROWS), :], rvm, local_sems.at[1])
        cp_pd = pltpu.make_async_copy(
            p_ref.at[pl.ds(drow0 + S0 * TILE, NS * TILE), :], pvd,
            local_sems.at[2])
        cp_rd = pltpu.make_async_copy(
            r_ref.at[pl.ds(drow0 + S0 * TILE, NS * TILE), :], rvd,
            local_sems.at[3])
        cp_p.start()
        cp_r.start()
        cp_pd.start()
        cp_rd.start()

        barrier = pltpu.get_barrier_semaphore()
        for nbr in (xp, yp, zp):
            pl.semaphore_signal(
                barrier, inc=1, device_id=nbr,
                device_id_type=pl.DeviceIdType.MESH,
            )
        pl.semaphore_wait(barrier, 3)

        cp_p.wait()
        yex = []
        for t in range(N_TILES):
            tl = pl.ds(t * TILE, TILE)
            r = exchange(pvm.at[tl, :], ycomm.at[tl, :], t, yp)
            r.start()
            yex.append(r)
        cp_pd.wait()
        ydx = []
        for t in range(S0, N_TILES):
            dl = pl.ds((t - S0) * TILE, TILE)
            r = exchange(pvd.at[dl, :], ycd.at[dl, :], N_TILES + t, yp)
            r.start()
            ydx.append(r)
        cp_r.wait()
        cp_rd.wait()

        LAG = 5

        fwds = []

        def service_relay(j):
            direct[j].wait()
            jl = pl.ds(j * TILE, TILE)
            got_row0 = xrow0 if j % 2 == 0 else zrow0
            got_rows = pl.ds(got_row0 + j * TILE, TILE)
            relay_peer = zp if j % 2 == 0 else xp
            st = pltpu.make_async_copy(
                vin.at[jl, :], out_ref.at[got_rows, :],
                local_sems.at[4 + N_TILES + NS + j])
            st.start()
            stores.append(st)
            fw = exchange(vin.at[jl, :], out_ref.at[got_rows, :],
                          4 * N_TILES + j, relay_peer)
            fw.start()
            fwds.append(fw)

        direct = []
        final = []
        stores = []
        for t in range(N_TILES):
            yex[t].wait()
            tl = pl.ds(t * TILE, TILE)
            vmine[tl, :] = rms_norm(
                pvm[tl, :] + ycomm[tl, :] + rvm[tl, :])
            mine_rows = pl.ds(row0 + t * TILE, TILE)
            st = pltpu.make_async_copy(
                vmine.at[tl, :], out_ref.at[mine_rows, :],
                local_sems.at[4 + t])
            st.start()
            stores.append(st)
            if t < S0:
                fwd_peer = xp if t % 2 == 0 else zp
                fin_peer = zp if t % 2 == 0 else xp
                s1 = exchange(vmine.at[tl, :], vin.at[tl, :], 2 * N_TILES + t,
                              fwd_peer)
                s2 = exchange(vmine.at[tl, :], out_ref.at[mine_rows, :],
                              3 * N_TILES + t, fin_peer)
            else:
                s1 = exchange(vmine.at[tl, :], out_ref.at[mine_rows, :],
                              2 * N_TILES + t, xp)
                s2 = exchange(vmine.at[tl, :], out_ref.at[mine_rows, :],
                              3 * N_TILES + t, zp)
            s1.start()
            s2.start()
            direct.append(s1)
            final.append(s2)
            if t >= LAG and t - LAG < S0:
                service_relay(t - LAG)

        for j in range(max(0, N_TILES - LAG), S0):
            service_relay(j)

        for i, r in enumerate(ydx):
            r.wait()
            t = S0 + i
            dl = pl.ds(i * TILE, TILE)
            vdiag[dl, :] = rms_norm(pvd[dl, :] + ycd[dl, :] + rvd[dl, :])
            st = pltpu.make_async_copy(
                vdiag.at[dl, :],
                out_ref.at[pl.ds(drow0 + t * TILE, TILE), :],
                local_sems.at[4 + N_TILES + i])
            st.start()
            stores.append(st)

        for t in range(S0, N_TILES):
            direct[t].wait()
        for t in range(N_TILES):
            final[t].wait()
        for fw in fwds:
            fw.wait()
        for st in stores:
            st.wait()

    return pl.pallas_call(
        body,
        out_shape=jax.ShapeDtypeStruct((M, D), jnp.float32),
        in_specs=[
            pl.BlockSpec(memory_space=pl.ANY),
            pl.BlockSpec(memory_space=pl.ANY),
            pl.BlockSpec(memory_space=pltpu.VMEM),
        ],
        out_specs=pl.BlockSpec(memory_space=pl.ANY),
        scratch_shapes=[
            pltpu.VMEM((ROWS, D), jnp.float32),
            pltpu.VMEM((ROWS, D), jnp.float32),
            pltpu.VMEM((ROWS, D), jnp.float32),
            pltpu.VMEM((NS * TILE, D), jnp.float32),
            pltpu.VMEM((NS * TILE, D), jnp.float32),
            pltpu.VMEM((NS * TILE, D), jnp.float32),
            pltpu.VMEM((ROWS, D), jnp.float32),
            pltpu.VMEM((NS * TILE, D), jnp.float32),
            pltpu.VMEM((S0 * TILE, D), jnp.float32),
            pltpu.SemaphoreType.DMA((5 * N_TILES,)),
            pltpu.SemaphoreType.DMA((5 * N_TILES,)),
            pltpu.SemaphoreType.DMA((4 + 2 * N_TILES,)),
        ],
        compiler_params=pltpu.CompilerParams(collective_id=0),
    )(partial2d, resid, gamma2d)


# device time: 32338 ns/iter; 1.8841x vs baseline; 1.0025x over previous
import jax
import jax.numpy as jnp
from jax import lax
from jax.experimental import pallas as pl
from jax.experimental.pallas import tpu as pltpu

M = 1024
D = 1024
N_GROUPS = 4
ROWS = M // N_GROUPS
N_TILES = 16
TILE = ROWS // N_TILES
S0 = 10
NS = N_TILES - S0
EPS = 1e-6


def kernel(partial, resid, gamma):
    partial2d = partial.reshape(M, D)
    gamma2d = gamma.reshape(1, D)

    def body(p_ref, r_ref, g_ref, out_ref, pvm, rvm, ycomm,
             pvd, rvd, ycd, vmine, vdiag, vin,
             send_sems, recv_sems, local_sems):
        my_x = lax.axis_index("x")
        my_y = lax.axis_index("y")
        my_z = lax.axis_index("z")
        xp = (1 - my_x, my_y, my_z)
        yp = (my_x, 1 - my_y, my_z)
        zp = (my_x, my_y, 1 - my_z)

        row0 = (2 * my_x + my_z) * ROWS
        xrow0 = (2 * (1 - my_x) + my_z) * ROWS
        zrow0 = (2 * my_x + (1 - my_z)) * ROWS
        drow0 = (2 * (1 - my_x) + (1 - my_z)) * ROWS

        def exchange(src, dst, sem, peer):
            return pltpu.make_async_remote_copy(
                src_ref=src, dst_ref=dst,
                send_sem=send_sems.at[sem], recv_sem=recv_sems.at[sem],
                device_id=peer, device_id_type=pl.DeviceIdType.MESH,
            )

        def rms_norm(acc):
            rms = jnp.sqrt(jnp.mean(acc * acc, axis=-1, keepdims=True) + EPS)
            return acc / rms * g_ref[...]

        cp_p = pltpu.make_async_copy(
            p_ref.at[pl.ds(row0, ROWS), :], pvm, local_sems.at[0])
        cp_r = pltpu.make_async_copy(
            r_ref.at[pl.ds(row0, ROWS), :], rvm, local_sems.at[1])
        cp_pd = pltpu.make_async_copy(
            p_ref.at[pl.ds(drow0 + S0 * TILE, NS * TILE), :], pvd,
            local_sems.at[2])
        cp_rd = pltpu.make_async_copy(
            r_ref.at[pl.ds(drow0 + S0 * TILE, NS * TILE), :], rvd,
            local_sems.at[3])
        cp_p.start()
        cp_r.start()
        cp_pd.start()
        cp_rd.start()

        barrier = pltpu.get_barrier_semaphore()
        for nbr in (xp, yp, zp):
            pl.semaphore_signal(
                barrier, inc=1, device_id=nbr,
                device_id_type=pl.DeviceIdType.MESH,
            )
        pl.semaphore_wait(barrier, 3)

        cp_p.wait()
        yex = []
        for t in range(N_TILES):
            tl = pl.ds(t * TILE, TILE)
            r = exchange(pvm.at[tl, :], ycomm.at[tl, :], t, yp)
            r.start()
            yex.append(r)
        cp_pd.wait()
        ydx = []
        for t in range(S0, N_TILES):
            dl = pl.ds((t - S0) * TILE, TILE)
            r = exchange(pvd.at[dl, :], ycd.at[dl, :], N_TILES + t, yp)
            r.start()
            ydx.append(r)
        cp_r.wait()
        cp_rd.wait()

        direct = []
        final = []
        stores = []
        for t in range(N_TILES):
            yex[t].wait()
            tl = pl.ds(t * TILE, TILE)
            vmine[tl, :] = rms_norm(
                pvm[tl, :] + ycomm[tl, :] + rvm[tl, :])
            mine_rows = pl.ds(row0 + t * TILE, TILE)
            st = pltpu.make_async_copy(
                vmine.at[tl, :], out_ref.at[mine_rows, :],
                local_sems.at[4 + t])
            st.start()
            stores.append(st)
            if t < S0:
                fwd_peer = xp if t % 2 == 0 else zp
                fin_peer = zp if t % 2 == 0 else xp
                s1 = exchange(vmine.at[tl, :], vin.at[tl, :], 2 * N_TILES + t,
                              fwd_peer)
                s2 = exchange(vmine.at[tl, :], out_ref.at[mine_rows, :],
                              3 * N_TILES + t, fin_peer)
            else:
                s1 = exchange(vmine.at[tl, :], out_ref.at[mine_rows, :],
                              2 * N_TILES + t, xp)
                s2 = exchange(vmine.at[tl, :], out_ref.at[mine_rows, :],
                              3 * N_TILES + t, zp)
            s1.start()
            s2.start()
            direct.append(s1)
            final.append(s2)

        fwds = []
        for t in range(S0):
            direct[t].wait()
            tl = pl.ds(t * TILE, TILE)
            got_row0 = xrow0 if t % 2 == 0 else zrow0
            got_rows = pl.ds(got_row0 + t * TILE, TILE)
            relay_peer = zp if t % 2 == 0 else xp
            st = pltpu.make_async_copy(
                vin.at[tl, :], out_ref.at[got_rows, :],
                local_sems.at[4 + N_TILES + NS + t])
            st.start()
            stores.append(st)
            fw = exchange(vin.at[tl, :], out_ref.at[got_rows, :],
                          4 * N_TILES + t, relay_peer)
            fw.start()
            fwds.append(fw)

        for i, r in enumerate(ydx):
            r.wait()
            t = S0 + i
            dl = pl.ds(i * TILE, TILE)
            vdiag[dl, :] = rms_norm(pvd[dl, :] + ycd[dl, :] + rvd[dl, :])
            st = pltpu.make_async_copy(
                vdiag.at[dl, :],
                out_ref.at[pl.ds(drow0 + t * TILE, TILE), :],
                local_sems.at[4 + N_TILES + i])
            st.start()
            stores.append(st)

        for t in range(S0, N_TILES):
            direct[t].wait()
        for t in range(N_TILES):
            final[t].wait()
        for fw in fwds:
            fw.wait()
        for st in stores:
            st.wait()

    return pl.pallas_call(
        body,
        out_shape=jax.ShapeDtypeStruct((M, D), jnp.float32),
        in_specs=[
            pl.BlockSpec(memory_space=pl.ANY),
            pl.BlockSpec(memory_space=pl.ANY),
            pl.BlockSpec(memory_space=pltpu.VMEM),
        ],
        out_specs=pl.BlockSpec(memory_space=pl.ANY),
        scratch_shapes=[
            pltpu.VMEM((ROWS, D), jnp.float32),
            pltpu.VMEM((ROWS, D), jnp.float32),
            pltpu.VMEM((ROWS, D), jnp.float32),
            pltpu.VMEM((NS * TILE, D), jnp.float32),
            pltpu.VMEM((NS * TILE, D), jnp.float32),
            pltpu.VMEM((NS * TILE, D), jnp.float32),
            pltpu.VMEM((ROWS, D), jnp.float32),
            pltpu.VMEM((NS * TILE, D), jnp.float32),
            pltpu.VMEM((S0 * TILE, D), jnp.float32),
            pltpu.SemaphoreType.DMA((5 * N_TILES,)),
            pltpu.SemaphoreType.DMA((5 * N_TILES,)),
            pltpu.SemaphoreType.DMA((4 + 2 * N_TILES,)),
        ],
        compiler_params=pltpu.CompilerParams(collective_id=0),
    )(partial2d, resid, gamma2d)
